# Initial kernel scaffold; baseline (speedup 1.0000x reference)
import jax
import jax.numpy as jnp
from jax.experimental import pallas as pl


def kernel(x, w, idx, W_bank, b_bank):
    raise NotImplementedError("write your pallas kernel here")


# speedup vs baseline: 3.6699x; 3.6699x over previous
import functools

import jax
import jax.numpy as jnp
from jax.experimental import pallas as pl
from jax.experimental.pallas import tpu as pltpu

_TT = 512


def _body(K, idx_ref, w_ref, x_ref, *refs):
    del idx_ref
    o_ref = refs[-1]
    W_refs = refs[:K]
    b_refs = refs[K : 2 * K]
    b = pl.program_id(0)
    x16 = x_ref[0].astype(jnp.bfloat16)
    acc = None
    for k in range(K):
        y = jnp.dot(
            x16,
            W_refs[k][0].astype(jnp.bfloat16),
            preferred_element_type=jnp.float32,
        )
        y = jnp.maximum(y + b_refs[k][0], 0.0) * w_ref[K * b + k]
        acc = y if acc is None else acc + y
    o_ref[0] = acc


def kernel(x, w, idx, W_bank, b_bank):
    B, T, D = x.shape
    K = idx.shape[1]
    idx_flat = idx.reshape(-1).astype(jnp.int32)
    w_flat = w.reshape(-1).astype(jnp.float32)

    def x_map(b, t, idx_ref, w_ref):
        return (b, t, 0)

    def W_map(k, b, t, idx_ref, w_ref):
        return (idx_ref[K * b + k], 0, 0)

    def b_map(k, b, t, idx_ref, w_ref):
        return (idx_ref[K * b + k], 0, 0)

    def o_map(b, t, idx_ref, w_ref):
        return (b, t, 0)

    in_specs = [pl.BlockSpec((1, _TT, D), x_map)]
    in_specs += [
        pl.BlockSpec((1, D, D), functools.partial(W_map, k)) for k in range(K)
    ]
    in_specs += [
        pl.BlockSpec((1, 1, D), functools.partial(b_map, k)) for k in range(K)
    ]

    grid_spec = pltpu.PrefetchScalarGridSpec(
        num_scalar_prefetch=2,
        grid=(B, T // _TT),
        in_specs=in_specs,
        out_specs=pl.BlockSpec((1, _TT, D), o_map),
    )
    out = pl.pallas_call(
        functools.partial(_body, K),
        grid_spec=grid_spec,
        out_shape=jax.ShapeDtypeStruct((B, T, D), jnp.float32),
        compiler_params=pltpu.CompilerParams(
            dimension_semantics=("parallel", "arbitrary"),
        ),
    )(idx_flat, w_flat, x, *([W_bank] * K), *([b_bank.reshape(-1, 1, D)] * K))
    return out


# speedup vs baseline: 3.8631x; 1.0527x over previous
import functools

import jax
import jax.numpy as jnp
from jax.experimental import pallas as pl
from jax.experimental.pallas import tpu as pltpu

_TT = 1024


def _body(K, idx_ref, w_ref, x_ref, *refs):
    del idx_ref
    o_ref = refs[-1]
    W_refs = refs[:K]
    b_refs = refs[K : 2 * K]
    b = pl.program_id(0)
    x16 = x_ref[0].astype(jnp.bfloat16)
    acc = None
    for k in range(K):
        y = jnp.dot(
            x16,
            W_refs[k][0].astype(jnp.bfloat16),
            preferred_element_type=jnp.float32,
        )
        y = jnp.maximum(y + b_refs[k][0], 0.0) * w_ref[K * b + k]
        acc = y if acc is None else acc + y
    o_ref[0] = acc


def kernel(x, w, idx, W_bank, b_bank):
    B, T, D = x.shape
    K = idx.shape[1]
    idx_flat = idx.reshape(-1).astype(jnp.int32)
    w_flat = w.reshape(-1).astype(jnp.float32)

    def x_map(b, t, idx_ref, w_ref):
        return (b, t, 0)

    def W_map(k, b, t, idx_ref, w_ref):
        return (idx_ref[K * b + k], 0, 0)

    def b_map(k, b, t, idx_ref, w_ref):
        return (idx_ref[K * b + k], 0, 0)

    def o_map(b, t, idx_ref, w_ref):
        return (b, t, 0)

    in_specs = [pl.BlockSpec((1, _TT, D), x_map)]
    in_specs += [
        pl.BlockSpec((1, D, D), functools.partial(W_map, k)) for k in range(K)
    ]
    in_specs += [
        pl.BlockSpec((1, 1, D), functools.partial(b_map, k)) for k in range(K)
    ]

    grid_spec = pltpu.PrefetchScalarGridSpec(
        num_scalar_prefetch=2,
        grid=(B, T // _TT),
        in_specs=in_specs,
        out_specs=pl.BlockSpec((1, _TT, D), o_map),
    )
    out = pl.pallas_call(
        functools.partial(_body, K),
        grid_spec=grid_spec,
        out_shape=jax.ShapeDtypeStruct((B, T, D), jnp.float32),
        compiler_params=pltpu.CompilerParams(
            dimension_semantics=("parallel", "arbitrary"),
        ),
    )(idx_flat, w_flat, x, *([W_bank] * K), *([b_bank.reshape(-1, 1, D)] * K))
    return out
